# baseline (speedup 1.0000x reference)
import functools

import jax
import jax.numpy as jnp
from jax import lax
from jax.experimental import pallas as pl
from jax.experimental.pallas import tpu as pltpu
from jax.experimental.pallas import tpu_sc as plsc

_EPS = 1e-05
_N = 100000
_C = 512
_NW = 32
_ROWS_W = _N // _NW
_CHUNK = 125
_NCHUNK = _ROWS_W // _CHUNK
_VPR = _C // 16


_GDN = lax.GatherDimensionNumbers(
    offset_dims=(), collapsed_slice_dims=(0,), start_index_map=(0,))


def _lane_shuffle(v, idx):
    return lax.gather(v, idx[:, None], _GDN, (1,),
                      mode=lax.GatherScatterMode.PROMISE_IN_BOUNDS)


def _allreduce_sum(v):
    lane = lax.iota(jnp.int32, 16)
    for k in (8, 4, 2, 1):
        v = v + _lane_shuffle(v, lane ^ k)
    return v


def _rsqrt_newton(x):
    i = lax.bitcast_convert_type(x, jnp.int32)
    i = jnp.int32(0x5F3759DF) - lax.shift_right_logical(i, 1)
    y = lax.bitcast_convert_type(i, jnp.float32)
    half = jnp.float32(0.5) * x
    for _ in range(3):
        y = y * (jnp.float32(1.5) - half * y * y)
    return y


def _tec_body(x_hbm, w_hbm, b_hbm, o_hbm, wv, bv, xb):
    wid = lax.axis_index("s") * 2 + lax.axis_index("c")
    base = wid * _ROWS_W
    pltpu.sync_copy(w_hbm, wv)
    pltpu.sync_copy(b_hbm, bv)

    def chunk_body(ci, _):
        row0 = base + ci * _CHUNK
        pltpu.sync_copy(x_hbm.at[pl.ds(row0 * _C, _CHUNK * _C)], xb)

        def row_body(r, _):
            off = r * _C

            def acc_body(j, carry):
                s, q = carry
                v = xb[pl.ds(off + j * 16, 16)]
                return s + v, q + v * v

            z = jnp.zeros((16,), jnp.float32)
            s, q = lax.fori_loop(0, _VPR, acc_body, (z, z))
            ssum = _allreduce_sum(s)
            qsum = _allreduce_sum(q)
            mean = ssum * jnp.float32(1.0 / _C)
            var = qsum * jnp.float32(1.0 / _C) - mean * mean
            inv = _rsqrt_newton(var + jnp.float32(_EPS))
            shift = mean * inv

            def norm_body(j, _):
                sl = pl.ds(off + j * 16, 16)
                wsl = pl.ds(j * 16, 16)
                v = xb[sl]
                xb[sl] = (v * inv - shift) * wv[wsl] + bv[wsl]
                return 0

            lax.fori_loop(0, _VPR, norm_body, 0)
            return 0

        lax.fori_loop(0, _CHUNK, row_body, 0)
        pltpu.sync_copy(xb, o_hbm.at[pl.ds(row0 * _C, _CHUNK * _C)])
        return 0

    lax.fori_loop(0, _NCHUNK, chunk_body, 0)


def kernel(input, weight, bias):
    n, c = input.shape
    mesh = plsc.VectorSubcoreMesh(core_axis_name="c", subcore_axis_name="s")
    k = functools.partial(
        pl.kernel,
        mesh=mesh,
        out_type=jax.ShapeDtypeStruct((n * c,), jnp.float32),
        scratch_types=[
            pltpu.VMEM((_C,), jnp.float32),
            pltpu.VMEM((_C,), jnp.float32),
            pltpu.VMEM((_CHUNK * _C,), jnp.float32),
        ],
    )(_tec_body)
    return k(input.reshape(n * c), weight, bias).reshape(n, c)


# speedup vs baseline: 1.3840x; 1.3840x over previous
import functools

import jax
import jax.numpy as jnp
from jax import lax
from jax.experimental import pallas as pl
from jax.experimental.pallas import tpu as pltpu
from jax.experimental.pallas import tpu_sc as plsc

_EPS = 1e-05
_N = 100000
_C = 512
_NW = 32
_ROWS_W = _N // _NW
_CHUNK = 125
_NCHUNK = _ROWS_W // _CHUNK
_VPR = _C // 16


_GDN = lax.GatherDimensionNumbers(
    offset_dims=(), collapsed_slice_dims=(0,), start_index_map=(0,))


def _lane_shuffle(v, idx):
    return lax.gather(v, idx[:, None], _GDN, (1,),
                      mode=lax.GatherScatterMode.PROMISE_IN_BOUNDS)


def _allreduce_sum(v):
    lane = lax.iota(jnp.int32, 16)
    for k in (8, 4, 2, 1):
        v = v + _lane_shuffle(v, lane ^ k)
    return v


def _rsqrt_newton(x):
    i = lax.bitcast_convert_type(x, jnp.int32)
    i = jnp.int32(0x5F3759DF) - lax.shift_right_logical(i, 1)
    y = lax.bitcast_convert_type(i, jnp.float32)
    half = jnp.float32(0.5) * x
    for _ in range(3):
        y = y * (jnp.float32(1.5) - half * y * y)
    return y


def _tec_body(x_hbm, w_hbm, b_hbm, o_hbm, wv, bv, xb):
    wid = lax.axis_index("s") * 2 + lax.axis_index("c")
    base = wid * _ROWS_W
    pltpu.sync_copy(w_hbm, wv)
    pltpu.sync_copy(b_hbm, bv)

    def chunk_body(ci, _):
        row0 = base + ci * _CHUNK
        pltpu.sync_copy(x_hbm.at[pl.ds(row0 * _C, _CHUNK * _C)], xb)

        def row_body(r, _):
            off = r * _C
            vs = [xb[pl.ds(off + j * 16, 16)] for j in range(_VPR)]
            s = z = jnp.zeros((16,), jnp.float32)
            q = z
            for v in vs:
                s = s + v
                q = q + v * v
            ssum = _allreduce_sum(s)
            qsum = _allreduce_sum(q)
            mean = ssum * jnp.float32(1.0 / _C)
            var = qsum * jnp.float32(1.0 / _C) - mean * mean
            inv = _rsqrt_newton(var + jnp.float32(_EPS))
            shift = mean * inv
            for j, v in enumerate(vs):
                wsl = pl.ds(j * 16, 16)
                xb[pl.ds(off + j * 16, 16)] = (
                    (v * inv - shift) * wv[wsl] + bv[wsl])
            return 0

        lax.fori_loop(0, _CHUNK, row_body, 0)
        pltpu.sync_copy(xb, o_hbm.at[pl.ds(row0 * _C, _CHUNK * _C)])
        return 0

    lax.fori_loop(0, _NCHUNK, chunk_body, 0)


def kernel(input, weight, bias):
    n, c = input.shape
    mesh = plsc.VectorSubcoreMesh(core_axis_name="c", subcore_axis_name="s")
    k = functools.partial(
        pl.kernel,
        mesh=mesh,
        out_type=jax.ShapeDtypeStruct((n * c,), jnp.float32),
        scratch_types=[
            pltpu.VMEM((_C,), jnp.float32),
            pltpu.VMEM((_C,), jnp.float32),
            pltpu.VMEM((_CHUNK * _C,), jnp.float32),
        ],
    )(_tec_body)
    return k(input.reshape(n * c), weight, bias).reshape(n, c)


# speedup vs baseline: 13.8504x; 10.0077x over previous
import jax
import jax.numpy as jnp
from jax.experimental import pallas as pl

_EPS = 1e-05
_N_NODES = 100000
_N_FEATURES = 512
_BLOCK_ROWS = 4000


def _stfn_block(x_ref, w_ref, b_ref, o_ref):
    x = x_ref[...]
    mean = jnp.mean(x, axis=1, keepdims=True)
    msq = jnp.mean(x * x, axis=1, keepdims=True)
    var = msq - mean * mean
    inv = jax.lax.rsqrt(var + _EPS)
    o_ref[...] = (x - mean) * (inv * w_ref[...]) + b_ref[...]


def kernel(input, weight, bias):
    n, c = input.shape
    grid = (n // _BLOCK_ROWS,)
    return pl.pallas_call(
        _stfn_block,
        grid=grid,
        in_specs=[
            pl.BlockSpec((_BLOCK_ROWS, c), lambda i: (i, 0)),
            pl.BlockSpec((1, c), lambda i: (0, 0)),
            pl.BlockSpec((1, c), lambda i: (0, 0)),
        ],
        out_specs=pl.BlockSpec((_BLOCK_ROWS, c), lambda i: (i, 0)),
        out_shape=jax.ShapeDtypeStruct((n, c), input.dtype),
    )(input, weight.reshape(1, c), bias.reshape(1, c))


# speedup vs baseline: 13.9366x; 1.0062x over previous
import jax
import jax.numpy as jnp
from jax.experimental import pallas as pl

_EPS = 1e-05
_N_NODES = 100000
_N_FEATURES = 512
_BLOCK_ROWS = 4000


def _stfn_block(x_ref, w_ref, b_ref, o_ref):
    x = x_ref[...]
    mean = jnp.mean(x, axis=1, keepdims=True)
    xc = x - mean
    var = jnp.mean(xc * xc, axis=1, keepdims=True)
    inv = jax.lax.rsqrt(var + _EPS)
    o_ref[...] = (xc * inv) * w_ref[...] + b_ref[...]


def kernel(input, weight, bias):
    n, c = input.shape
    grid = (n // _BLOCK_ROWS,)
    return pl.pallas_call(
        _stfn_block,
        grid=grid,
        in_specs=[
            pl.BlockSpec((_BLOCK_ROWS, c), lambda i: (i, 0)),
            pl.BlockSpec((1, c), lambda i: (0, 0)),
            pl.BlockSpec((1, c), lambda i: (0, 0)),
        ],
        out_specs=pl.BlockSpec((_BLOCK_ROWS, c), lambda i: (i, 0)),
        out_shape=jax.ShapeDtypeStruct((n, c), input.dtype),
    )(input, weight.reshape(1, c), bias.reshape(1, c))
